# baseline (speedup 1.0000x reference)
import functools

import jax
import jax.numpy as jnp
import numpy as np
from jax.experimental import pallas as pl
from jax.sharding import Mesh, PartitionSpec as P

_SCALE = 0.044194173824159216
_TEMPERATURE = 0.2
_C = -_SCALE / _TEMPERATURE

_BN = 4096


def _dist_kernel(q_ref, k_ref, o_ref):
    q = q_ref[...].astype(jnp.bfloat16)
    k = k_ref[...].astype(jnp.bfloat16)
    acc = jax.lax.dot_general(
        q, k, (((1,), (1,)), ((), ())),
        preferred_element_type=jnp.float32)
    o_ref[...] = acc * _C


def _local(query, key):
    m, d = query.shape
    n = key.shape[0]
    return pl.pallas_call(
        _dist_kernel,
        grid=(n // _BN,),
        in_specs=[
            pl.BlockSpec((m, d), lambda i: (0, 0)),
            pl.BlockSpec((_BN, d), lambda i: (i, 0)),
        ],
        out_specs=pl.BlockSpec((m, _BN), lambda i: (0, i)),
        out_shape=jax.ShapeDtypeStruct((m, n), jnp.float32),
    )(query, key)


@jax.jit
def kernel(query, key):
    devs = jax.devices()
    if len(devs) >= 2:
        mesh = Mesh(np.array(devs[:2]), ("x",))
        f = jax.shard_map(
            _local, mesh=mesh,
            in_specs=(P(None, None), P("x", None)),
            out_specs=P(None, "x"), check_vma=False)
        return f(query, key)
    return _local(query, key)


# speedup vs baseline: 4.4485x; 4.4485x over previous
import jax
import jax.numpy as jnp
from jax.experimental import pallas as pl

_SCALE = 0.044194173824159216
_TEMPERATURE = 0.2
_C = -_SCALE / _TEMPERATURE

_BN = 4096


def _dist_kernel(q_ref, k_ref, o_ref):
    q = (q_ref[...] * _C).astype(jnp.bfloat16)
    k = k_ref[...].astype(jnp.bfloat16)
    o_ref[...] = jax.lax.dot_general(
        q, k, (((1,), (1,)), ((), ())),
        preferred_element_type=jnp.float32)


@jax.jit
def kernel(query, key):
    m, d = query.shape
    n = key.shape[0]
    return pl.pallas_call(
        _dist_kernel,
        grid=(n // _BN,),
        in_specs=[
            pl.BlockSpec((m, d), lambda i: (0, 0)),
            pl.BlockSpec((_BN, d), lambda i: (i, 0)),
        ],
        out_specs=pl.BlockSpec((m, _BN), lambda i: (0, i)),
        out_shape=jax.ShapeDtypeStruct((m, n), jnp.float32),
    )(query, key)
